# baseline (speedup 1.0000x reference)
import jax
import jax.numpy as jnp
from jax import lax
from jax.experimental import pallas as pl
from jax.experimental.pallas import tpu as pltpu
from jax.experimental.pallas import tpu_sc as plsc

_Q = 16
_K = 4096
_D = 768
_C = 256
_CS = 32
_HALF = _K // 2
_N_STRUCT = 17
_N_BC0 = 5
_N_BC1 = 9


def _body(emb_hbm, out_hbm, zidx_v, sidx_v, rows_a, rows_b,
          bcast_sh, gsem, bsem, wsem):
    cid = lax.axis_index("c")
    sid = lax.axis_index("s")
    i = sid
    half = (sid + cid) & 1
    base = i * _K
    s_end = ((i + 513 + 7) >> 3) << 3

    zidx_v[...] = jnp.zeros((16,), jnp.int32)
    pltpu.async_copy(emb_hbm.at[zidx_v], rows_a.at[pl.ds(0, 16)], gsem).wait()
    pltpu.sync_copy(
        rows_a.at[pl.ds(0, 16)],
        bcast_sh.at[pl.ds(pl.multiple_of(sid * 16, 8), 16)])

    plsc.subcore_barrier()

    @pl.when(half == 0)
    def _():
        bufs = (rows_a, rows_b)

        def fill_idx(k):
            j0 = jnp.minimum(_CS * k, s_end - _CS)
            top = i + 512 - j0
            for q in range(_CS // 16):
                sidx_v[pl.ds(q * 16, 16)] = jnp.maximum(
                    (top - q * 16) - lax.iota(jnp.int32, 16), 0)
            return j0

        j0_prev = fill_idx(0)
        pltpu.async_copy(emb_hbm.at[sidx_v], bufs[0], gsem).wait()
        for k in range(1, _N_STRUCT + 1):
            wr = pltpu.async_copy(
                bufs[(k - 1) % 2],
                out_hbm.at[pl.ds(pl.multiple_of(base + j0_prev, 8), _CS)],
                wsem)
            if k < _N_STRUCT:
                j0_prev = fill_idx(k)
                pltpu.async_copy(emb_hbm.at[sidx_v], bufs[k % 2], gsem).wait()
            wr.wait()

    pend = []
    for k in range(_N_BC0):
        j0 = jnp.where(
            half == 0,
            s_end + _C * k,
            jnp.minimum(s_end + _C * (_N_BC0 + k), _K - _C),
        )
        pend.append(
            pltpu.async_copy(
                bcast_sh,
                out_hbm.at[pl.ds(pl.multiple_of(base + j0, 8), _C)], bsem))

    @pl.when(half == 1)
    def _():
        extra = [
            pltpu.async_copy(
                bcast_sh,
                out_hbm.at[pl.ds(
                    pl.multiple_of(
                        base + jnp.minimum(
                            s_end + _C * (_N_BC0 + k), _K - _C), 8),
                    _C)], bsem)
            for k in range(_N_BC0, _N_BC1)
        ]
        for p in extra:
            p.wait()

    for p in pend:
        p.wait()


@jax.jit
def _rpe(emb_weight):
    mesh = plsc.VectorSubcoreMesh(core_axis_name="c", subcore_axis_name="s")
    run = pl.kernel(
        _body,
        out_type=jax.ShapeDtypeStruct((_Q * _K, _D), jnp.float32),
        mesh=mesh,
        scratch_types=[
            pltpu.VMEM((16,), jnp.int32),
            pltpu.VMEM((_CS,), jnp.int32),
            pltpu.VMEM((_CS, _D), jnp.float32),
            pltpu.VMEM((_CS, _D), jnp.float32),
            pltpu.VMEM_SHARED((_C, _D), jnp.float32),
            pltpu.SemaphoreType.DMA,
            pltpu.SemaphoreType.DMA,
            pltpu.SemaphoreType.DMA,
        ],
    )
    return run(emb_weight).reshape(_Q, _K, _D)


def kernel(q_len, k_len, emb_weight):
    return _rpe(emb_weight)


# speedup vs baseline: 1.1777x; 1.1777x over previous
import jax
import jax.numpy as jnp
from jax import lax
from jax.experimental import pallas as pl
from jax.experimental.pallas import tpu as pltpu
from jax.experimental.pallas import tpu_sc as plsc

_Q = 16
_K = 4096
_D = 768
_C = 256
_CB = 64
_CS = 32
_HALF = _K // 2
_N_STRUCT = 17
_N_BC0 = 5
_N_BC1 = 9


def _body(emb_hbm, out_hbm, idx_v, sidx_v, bcast_v, rows_a, rows_b,
          bcast_sh, gsem, bsem, wsem):
    cid = lax.axis_index("c")
    sid = lax.axis_index("s")
    i = sid
    half = (sid + cid) & 1
    base = i * _K
    s_end = ((i + 513 + 7) >> 3) << 3

    @pl.when(sid == 0)
    def _():
        for q in range(4):
            idx_v[pl.ds(q * 16, 16)] = jnp.zeros((16,), jnp.int32)
        pltpu.async_copy(emb_hbm.at[idx_v], bcast_v, gsem).wait()
        for r in range(_C // _CB):
            pltpu.sync_copy(bcast_v, bcast_sh.at[pl.ds(r * _CB, _CB)])

    plsc.subcore_barrier()

    @pl.when(half == 0)
    def _():
        bufs = (rows_a, rows_b)

        def fill_idx(k):
            j0 = jnp.minimum(_CS * k, s_end - _CS)
            top = i + 512 - j0
            for q in range(_CS // 16):
                sidx_v[pl.ds(q * 16, 16)] = jnp.maximum(
                    (top - q * 16) - lax.iota(jnp.int32, 16), 0)
            return j0

        j0_prev = fill_idx(0)
        pltpu.async_copy(emb_hbm.at[sidx_v], bufs[0], gsem).wait()
        for k in range(1, _N_STRUCT + 1):
            wr = pltpu.async_copy(
                bufs[(k - 1) % 2],
                out_hbm.at[pl.ds(pl.multiple_of(base + j0_prev, 8), _CS)],
                wsem)
            if k < _N_STRUCT:
                j0_prev = fill_idx(k)
                pltpu.async_copy(emb_hbm.at[sidx_v], bufs[k % 2], gsem).wait()
            wr.wait()

    pend = []
    for k in range(_N_BC0):
        j0 = jnp.where(
            half == 0,
            s_end + _C * k,
            jnp.minimum(s_end + _C * (_N_BC0 + k), _K - _C),
        )
        pend.append(
            pltpu.async_copy(
                bcast_sh,
                out_hbm.at[pl.ds(pl.multiple_of(base + j0, 8), _C)], bsem))

    @pl.when(half == 1)
    def _():
        extra = [
            pltpu.async_copy(
                bcast_sh,
                out_hbm.at[pl.ds(
                    pl.multiple_of(
                        base + jnp.minimum(
                            s_end + _C * (_N_BC0 + k), _K - _C), 8),
                    _C)], bsem)
            for k in range(_N_BC0, _N_BC1)
        ]
        for p in extra:
            p.wait()

    for p in pend:
        p.wait()


@jax.jit
def _rpe(emb_weight):
    mesh = plsc.VectorSubcoreMesh(core_axis_name="c", subcore_axis_name="s")
    run = pl.kernel(
        _body,
        out_type=jax.ShapeDtypeStruct((_Q * _K, _D), jnp.float32),
        mesh=mesh,
        scratch_types=[
            pltpu.VMEM((_CB,), jnp.int32),
            pltpu.VMEM((_CS,), jnp.int32),
            pltpu.VMEM((_CB, _D), jnp.float32),
            pltpu.VMEM((_CS, _D), jnp.float32),
            pltpu.VMEM((_CS, _D), jnp.float32),
            pltpu.VMEM_SHARED((_C, _D), jnp.float32),
            pltpu.SemaphoreType.DMA,
            pltpu.SemaphoreType.DMA,
            pltpu.SemaphoreType.DMA,
        ],
    )
    return run(emb_weight).reshape(_Q, _K, _D)


def kernel(q_len, k_len, emb_weight):
    return _rpe(emb_weight)
